# Initial kernel scaffold; baseline (speedup 1.0000x reference)
import jax
import jax.numpy as jnp
from jax.experimental import pallas as pl


def kernel(x, edge_index, W, a, Wo, bo):
    raise NotImplementedError("write your pallas kernel here")


# speedup vs baseline: 93.6747x; 93.6747x over previous
import functools

import jax
import jax.numpy as jnp
from jax import lax
from jax.experimental import pallas as pl
from jax.experimental.pallas import tpu as pltpu
from jax.experimental.pallas import tpu_sc as plsc

_NC = 2
_NS = 16
_C = 80


def _proj_body(x_ref, wc_ref, asrc_ref, adst_ref, h_ref, ts_ref, td_ref, m_ref):
    h = jnp.dot(x_ref[...], wc_ref[...], preferred_element_type=jnp.float32)
    h_ref[...] = h
    ts = jnp.dot(h, asrc_ref[...], preferred_element_type=jnp.float32)
    td = jnp.dot(h, adst_ref[...], preferred_element_type=jnp.float32)
    ts_ref[...] = ts
    td_ref[...] = td
    blk = jnp.concatenate(
        [jnp.max(ts, axis=0)[None, :], jnp.max(td, axis=0)[None, :]], axis=0)

    @pl.when(pl.program_id(0) == 0)
    def _():
        m_ref[...] = blk

    @pl.when(pl.program_id(0) != 0)
    def _():
        m_ref[...] = jnp.maximum(m_ref[...], blk)


def _epilogue_body(o_ref, s_ref, bexp_ref, wo_ref, bo_ref, y_ref):
    o2 = o_ref[...]
    s2 = s_ref[...]
    s = s2[0] + s2[1]
    den = jnp.dot(s, bexp_ref[...], preferred_element_type=jnp.float32)
    o = (o2[0] + o2[1]) / (den + 1e-16)
    y = jnp.dot(o, wo_ref[...], preferred_element_type=jnp.float32)
    y = y + bo_ref[...]
    y_ref[...] = jnp.where(y > 0.0, y, jnp.exp(jnp.minimum(y, 0.0)) - 1.0)


def _edge_body(n_nodes, n_edges, heads,
               src_hbm, dst_hbm, h_hbm, ts_hbm, td_hbm, mpat_hbm,
               o_out, s_out,
               sidx_v, didx_v, gsrc_v, gdst_v, p_v, rows_v, mpat_v,
               o_sh, s_sh, sem_r, sem_s, sem_d):
    d = heads * 16
    cid = lax.axis_index("c")
    sid = lax.axis_index("s")
    wid = cid * _NS + sid
    ew = n_edges // (_NC * _NS)
    nchunk = ew // _C
    rows_main = (n_nodes // (8 * _NS)) * 8
    tail = n_nodes - _NS * rows_main

    zv = jnp.zeros((16,), jnp.float32)

    def _zb(r, _):
        def _zc(c, _):
            rows_v[r, pl.ds(c * 16, 16)] = zv
            return 0
        lax.fori_loop(0, d // 16, _zc, 0)
        p_v[r, :] = zv
        return 0

    lax.fori_loop(0, _C, _zb, 0)

    nz_full = rows_main // _C
    z_rem = rows_main - nz_full * _C
    for r in range(nz_full):
        pltpu.sync_copy(rows_v, o_sh.at[pl.ds(sid * rows_main + r * _C, _C)])
        pltpu.sync_copy(p_v, s_sh.at[pl.ds(sid * rows_main + r * _C, _C)])
    if z_rem:
        pltpu.sync_copy(rows_v.at[pl.ds(0, z_rem)],
                        o_sh.at[pl.ds(sid * rows_main + nz_full * _C, z_rem)])
        pltpu.sync_copy(p_v.at[pl.ds(0, z_rem)],
                        s_sh.at[pl.ds(sid * rows_main + nz_full * _C, z_rem)])

    @pl.when(sid == _NS - 1)
    def _():
        base = _NS * rows_main
        pltpu.sync_copy(rows_v.at[pl.ds(0, tail)], o_sh.at[pl.ds(base, tail)])
        pltpu.sync_copy(p_v.at[pl.ds(0, tail)], s_sh.at[pl.ds(base, tail)])

    pltpu.sync_copy(mpat_hbm, mpat_v)
    plsc.subcore_barrier()

    mv = mpat_v[...]

    def _chunk(g, _):
        base = wid * ew + g * _C
        pltpu.sync_copy(src_hbm.at[pl.ds(base, _C)], sidx_v)
        pltpu.sync_copy(dst_hbm.at[pl.ds(base, _C)], didx_v)
        cp_r = pltpu.async_copy(h_hbm.at[sidx_v], rows_v, sem_r)
        cp_s = pltpu.async_copy(ts_hbm.at[sidx_v], gsrc_v, sem_s)
        cp_d = pltpu.async_copy(td_hbm.at[didx_v], gdst_v, sem_d)
        cp_s.wait()
        cp_d.wait()

        def _pb(i, _):
            e = gsrc_v[i, :] + gdst_v[i, :]
            e = jnp.where(e < 0.0, e * 0.2, e)
            p_v[i, :] = jnp.exp(e - mv)
            return 0

        lax.fori_loop(0, _C, _pb, 0)
        cp_r.wait()

        def _sb(i, _):
            pv = p_v[i, :]
            for h in range(heads):
                b = lax.broadcast(pv[h], (16,))
                rows_v[i, pl.ds(h * 16, 16)] = rows_v[i, pl.ds(h * 16, 16)] * b
            return 0

        lax.fori_loop(0, _C, _sb, 0)
        pltpu.sync_copy(p_v, s_sh.at[didx_v], add=True)
        pltpu.sync_copy(rows_v, o_sh.at[didx_v], add=True)
        return 0

    lax.fori_loop(0, nchunk, _chunk, 0)
    plsc.subcore_barrier()

    row0 = sid * rows_main
    pltpu.sync_copy(o_sh.at[pl.ds(row0, rows_main)],
                    o_out.at[cid, pl.ds(row0, rows_main)])
    pltpu.sync_copy(s_sh.at[pl.ds(row0, rows_main)],
                    s_out.at[cid, pl.ds(row0, rows_main)])

    @pl.when(sid == _NS - 1)
    def _():
        base = _NS * rows_main
        pltpu.sync_copy(o_sh.at[pl.ds(base, tail)],
                        o_out.at[cid, pl.ds(base, tail)])
        pltpu.sync_copy(s_sh.at[pl.ds(base, tail)],
                        s_out.at[cid, pl.ds(base, tail)])


def kernel(x, edge_index, W, a, Wo, bo):
    n, din = x.shape
    heads, _, hid = W.shape
    e = edge_index.shape[1]
    d = heads * hid
    out_dim = Wo.shape[1]
    f32 = jnp.float32

    wc = jnp.transpose(W, (1, 0, 2)).reshape(din, d)
    eye = jnp.eye(heads, dtype=f32)
    a_src = (a[:, :hid][:, :, None] * eye[:, None, :]).reshape(d, heads)
    a_dst = (a[:, hid:][:, :, None] * eye[:, None, :]).reshape(d, heads)
    asrc2 = jnp.concatenate([a_src, a_src], axis=1)
    adst2 = jnp.concatenate([a_dst, a_dst], axis=1)
    bexp = jnp.concatenate(
        [jnp.kron(eye, jnp.ones((1, hid), f32)), jnp.zeros((heads, d), f32)],
        axis=0)
    bo2 = bo.reshape(1, out_dim)

    bn = 1000
    grid = n // bn
    h_mat, tsrc, tdst, mstats = pl.pallas_call(
        _proj_body,
        grid=(grid,),
        in_specs=[
            pl.BlockSpec((bn, din), lambda i: (i, 0)),
            pl.BlockSpec((din, d), lambda i: (0, 0)),
            pl.BlockSpec((d, 16), lambda i: (0, 0)),
            pl.BlockSpec((d, 16), lambda i: (0, 0)),
        ],
        out_specs=[
            pl.BlockSpec((bn, d), lambda i: (i, 0)),
            pl.BlockSpec((bn, 16), lambda i: (i, 0)),
            pl.BlockSpec((bn, 16), lambda i: (i, 0)),
            pl.BlockSpec((2, 16), lambda i: (0, 0)),
        ],
        out_shape=[
            jax.ShapeDtypeStruct((n, d), f32),
            jax.ShapeDtypeStruct((n, 16), f32),
            jax.ShapeDtypeStruct((n, 16), f32),
            jax.ShapeDtypeStruct((2, 16), f32),
        ],
    )(x, wc, asrc2, adst2)

    msum = mstats[0] + mstats[1]
    mpat = jnp.where(msum > 0.0, msum, 0.2 * msum)

    src = edge_index[0]
    dst = edge_index[1]

    edge_kernel = pl.kernel(
        functools.partial(_edge_body, n, e, heads),
        out_type=[
            jax.ShapeDtypeStruct((_NC, n, d), f32),
            jax.ShapeDtypeStruct((_NC, n, 16), f32),
        ],
        mesh=plsc.VectorSubcoreMesh(core_axis_name="c", subcore_axis_name="s"),
        compiler_params=pltpu.CompilerParams(use_tc_tiling_on_sc=False),
        scratch_types=[
            pltpu.VMEM((_C,), jnp.int32),
            pltpu.VMEM((_C,), jnp.int32),
            pltpu.VMEM((_C, 16), f32),
            pltpu.VMEM((_C, 16), f32),
            pltpu.VMEM((_C, 16), f32),
            pltpu.VMEM((_C, d), f32),
            pltpu.VMEM((16,), f32),
            pltpu.VMEM_SHARED((n, d), f32),
            pltpu.VMEM_SHARED((n, 16), f32),
            pltpu.SemaphoreType.DMA,
            pltpu.SemaphoreType.DMA,
            pltpu.SemaphoreType.DMA,
        ],
    )
    o2, s2 = edge_kernel(src, dst, h_mat, tsrc, tdst, mpat)

    y = pl.pallas_call(
        _epilogue_body,
        grid=(grid,),
        in_specs=[
            pl.BlockSpec((2, bn, d), lambda i: (0, i, 0)),
            pl.BlockSpec((2, bn, 16), lambda i: (0, i, 0)),
            pl.BlockSpec((16, d), lambda i: (0, 0)),
            pl.BlockSpec((d, out_dim), lambda i: (0, 0)),
            pl.BlockSpec((1, out_dim), lambda i: (0, 0)),
        ],
        out_specs=pl.BlockSpec((bn, out_dim), lambda i: (i, 0)),
        out_shape=jax.ShapeDtypeStruct((n, out_dim), f32),
    )(o2, s2, bexp, Wo, bo2)
    return y


# speedup vs baseline: 119.8870x; 1.2798x over previous
import functools

import jax
import jax.numpy as jnp
from jax import lax
from jax.experimental import pallas as pl
from jax.experimental.pallas import tpu as pltpu
from jax.experimental.pallas import tpu_sc as plsc

_NC = 2
_NS = 16
_C = 80


def _proj_body(x_ref, wc_ref, asrc_ref, adst_ref, h_ref, ts_ref, td_ref, m_ref):
    h = jnp.dot(x_ref[...], wc_ref[...], preferred_element_type=jnp.float32)
    h_ref[...] = h
    ts = jnp.dot(h, asrc_ref[...], preferred_element_type=jnp.float32)
    td = jnp.dot(h, adst_ref[...], preferred_element_type=jnp.float32)
    ts_ref[...] = ts
    td_ref[...] = td
    blk = jnp.concatenate(
        [jnp.max(ts, axis=0)[None, :], jnp.max(td, axis=0)[None, :]], axis=0)

    @pl.when(pl.program_id(0) == 0)
    def _():
        m_ref[...] = blk

    @pl.when(pl.program_id(0) != 0)
    def _():
        m_ref[...] = jnp.maximum(m_ref[...], blk)


def _epilogue_body(o_ref, s_ref, bexp_ref, wo_ref, bo_ref, y_ref):
    o2 = o_ref[...]
    s2 = s_ref[...]
    s = s2[0] + s2[1]
    den = jnp.dot(s, bexp_ref[...], preferred_element_type=jnp.float32)
    o = (o2[0] + o2[1]) / (den + 1e-16)
    y = jnp.dot(o, wo_ref[...], preferred_element_type=jnp.float32)
    y = y + bo_ref[...]
    y_ref[...] = jnp.where(y > 0.0, y, jnp.exp(jnp.minimum(y, 0.0)) - 1.0)


def _edge_body(n_nodes, n_edges, heads,
               src_hbm, dst_hbm, h_hbm, ts_hbm, td_hbm, mpat_hbm,
               o_out, s_out,
               sidx0, didx0, gsrc0, gdst0, p0, rows0,
               sidx1, didx1, gsrc1, gdst1, p1, rows1,
               mpat_v, o_sh, s_sh,
               gr0, gs0, gd0, gr1, gs1, gd1, sp0, so0, sp1, so1):
    buf_a = (sidx0, didx0, gsrc0, gdst0, p0, rows0, gr0, gs0, gd0, sp0, so0)
    buf_b = (sidx1, didx1, gsrc1, gdst1, p1, rows1, gr1, gs1, gd1, sp1, so1)
    sidx_v, didx_v, p_v, rows_v = sidx0, didx0, p0, rows0
    d = heads * 16
    cid = lax.axis_index("c")
    sid = lax.axis_index("s")
    wid = cid * _NS + sid
    ew = n_edges // (_NC * _NS)
    nchunk = ew // _C
    rows_main = (n_nodes // (8 * _NS)) * 8
    tail = n_nodes - _NS * rows_main

    zv = jnp.zeros((16,), jnp.float32)

    def _zb(r, _):
        def _zc(c, _):
            rows_v[r, pl.ds(c * 16, 16)] = zv
            return 0
        lax.fori_loop(0, d // 16, _zc, 0)
        p_v[r, :] = zv
        return 0

    lax.fori_loop(0, _C, _zb, 0)

    nz_full = rows_main // _C
    z_rem = rows_main - nz_full * _C
    for r in range(nz_full):
        pltpu.sync_copy(rows_v, o_sh.at[pl.ds(sid * rows_main + r * _C, _C)])
        pltpu.sync_copy(p_v, s_sh.at[pl.ds(sid * rows_main + r * _C, _C)])
    if z_rem:
        pltpu.sync_copy(rows_v.at[pl.ds(0, z_rem)],
                        o_sh.at[pl.ds(sid * rows_main + nz_full * _C, z_rem)])
        pltpu.sync_copy(p_v.at[pl.ds(0, z_rem)],
                        s_sh.at[pl.ds(sid * rows_main + nz_full * _C, z_rem)])

    @pl.when(sid == _NS - 1)
    def _():
        base = _NS * rows_main
        pltpu.sync_copy(rows_v.at[pl.ds(0, tail)], o_sh.at[pl.ds(base, tail)])
        pltpu.sync_copy(p_v.at[pl.ds(0, tail)], s_sh.at[pl.ds(base, tail)])

    pltpu.sync_copy(mpat_hbm, mpat_v)
    plsc.subcore_barrier()

    mv = mpat_v[...]

    def _issue_gather(g, b):
        sidx, didx, gsrc, gdst, _, rows, gr, gs, gd, _, _ = b
        base = wid * ew + g * _C
        pltpu.sync_copy(src_hbm.at[pl.ds(base, _C)], sidx)
        pltpu.sync_copy(dst_hbm.at[pl.ds(base, _C)], didx)
        pltpu.async_copy(h_hbm.at[sidx], rows, gr)
        pltpu.async_copy(ts_hbm.at[sidx], gsrc, gs)
        pltpu.async_copy(td_hbm.at[didx], gdst, gd)

    def _wait_gather(b):
        sidx, didx, gsrc, gdst, _, rows, gr, gs, gd, _, _ = b
        pltpu.make_async_copy(h_hbm.at[sidx], rows, gr).wait()
        pltpu.make_async_copy(ts_hbm.at[sidx], gsrc, gs).wait()
        pltpu.make_async_copy(td_hbm.at[didx], gdst, gd).wait()

    def _issue_scatter(b):
        _, didx, _, _, p, rows, _, _, _, sp, so = b
        pltpu.async_copy(p, s_sh.at[didx], sp, add=True)
        pltpu.async_copy(rows, o_sh.at[didx], so, add=True)

    def _wait_scatter(b):
        _, didx, _, _, p, rows, _, _, _, sp, so = b
        pltpu.make_async_copy(p, s_sh.at[didx], sp).wait()
        pltpu.make_async_copy(rows, o_sh.at[didx], so).wait()

    def _compute(b):
        _, _, gsrc, gdst, p, rows, _, _, _, _, _ = b

        def _pb(i, _):
            e = gsrc[i, :] + gdst[i, :]
            e = jnp.where(e < 0.0, e * 0.2, e)
            p[i, :] = jnp.exp(e - mv)
            return 0

        lax.fori_loop(0, _C, _pb, 0)

        def _sb(i, _):
            pv = p[i, :]
            for h in range(heads):
                bc = lax.broadcast(pv[h], (16,))
                rows[i, pl.ds(h * 16, 16)] = rows[i, pl.ds(h * 16, 16)] * bc
            return 0

        lax.fori_loop(0, _C, _sb, 0)

    def _half(g, x, y):
        @pl.when(g > 0)
        def _():
            _wait_scatter(y)

        @pl.when(g + 1 < nchunk)
        def _():
            _issue_gather(g + 1, y)

        _wait_gather(x)
        _compute(x)
        _issue_scatter(x)

    _issue_gather(0, buf_a)

    def _body(i, _):
        _half(2 * i, buf_a, buf_b)
        _half(2 * i + 1, buf_b, buf_a)
        return 0

    lax.fori_loop(0, nchunk // 2, _body, 0)
    if nchunk % 2:
        _half(nchunk - 1, buf_a, buf_b)
        _wait_scatter(buf_a)
    else:
        _wait_scatter(buf_b)
    plsc.subcore_barrier()

    row0 = sid * rows_main
    pltpu.sync_copy(o_sh.at[pl.ds(row0, rows_main)],
                    o_out.at[cid, pl.ds(row0, rows_main)])
    pltpu.sync_copy(s_sh.at[pl.ds(row0, rows_main)],
                    s_out.at[cid, pl.ds(row0, rows_main)])

    @pl.when(sid == _NS - 1)
    def _():
        base = _NS * rows_main
        pltpu.sync_copy(o_sh.at[pl.ds(base, tail)],
                        o_out.at[cid, pl.ds(base, tail)])
        pltpu.sync_copy(s_sh.at[pl.ds(base, tail)],
                        s_out.at[cid, pl.ds(base, tail)])


def kernel(x, edge_index, W, a, Wo, bo):
    n, din = x.shape
    heads, _, hid = W.shape
    e = edge_index.shape[1]
    d = heads * hid
    out_dim = Wo.shape[1]
    f32 = jnp.float32

    wc = jnp.transpose(W, (1, 0, 2)).reshape(din, d)
    eye = jnp.eye(heads, dtype=f32)
    a_src = (a[:, :hid][:, :, None] * eye[:, None, :]).reshape(d, heads)
    a_dst = (a[:, hid:][:, :, None] * eye[:, None, :]).reshape(d, heads)
    asrc2 = jnp.concatenate([a_src, a_src], axis=1)
    adst2 = jnp.concatenate([a_dst, a_dst], axis=1)
    bexp = jnp.concatenate(
        [jnp.kron(eye, jnp.ones((1, hid), f32)), jnp.zeros((heads, d), f32)],
        axis=0)
    bo2 = bo.reshape(1, out_dim)

    bn = 1000
    grid = n // bn
    h_mat, tsrc, tdst, mstats = pl.pallas_call(
        _proj_body,
        grid=(grid,),
        in_specs=[
            pl.BlockSpec((bn, din), lambda i: (i, 0)),
            pl.BlockSpec((din, d), lambda i: (0, 0)),
            pl.BlockSpec((d, 16), lambda i: (0, 0)),
            pl.BlockSpec((d, 16), lambda i: (0, 0)),
        ],
        out_specs=[
            pl.BlockSpec((bn, d), lambda i: (i, 0)),
            pl.BlockSpec((bn, 16), lambda i: (i, 0)),
            pl.BlockSpec((bn, 16), lambda i: (i, 0)),
            pl.BlockSpec((2, 16), lambda i: (0, 0)),
        ],
        out_shape=[
            jax.ShapeDtypeStruct((n, d), f32),
            jax.ShapeDtypeStruct((n, 16), f32),
            jax.ShapeDtypeStruct((n, 16), f32),
            jax.ShapeDtypeStruct((2, 16), f32),
        ],
    )(x, wc, asrc2, adst2)

    msum = mstats[0] + mstats[1]
    mpat = jnp.where(msum > 0.0, msum, 0.2 * msum)

    src = edge_index[0]
    dst = edge_index[1]

    edge_kernel = pl.kernel(
        functools.partial(_edge_body, n, e, heads),
        out_type=[
            jax.ShapeDtypeStruct((_NC, n, d), f32),
            jax.ShapeDtypeStruct((_NC, n, 16), f32),
        ],
        mesh=plsc.VectorSubcoreMesh(core_axis_name="c", subcore_axis_name="s"),
        compiler_params=pltpu.CompilerParams(use_tc_tiling_on_sc=False),
        scratch_types=(
            [
                pltpu.VMEM((_C,), jnp.int32),
                pltpu.VMEM((_C,), jnp.int32),
                pltpu.VMEM((_C, 16), f32),
                pltpu.VMEM((_C, 16), f32),
                pltpu.VMEM((_C, 16), f32),
                pltpu.VMEM((_C, d), f32),
            ] * 2
            + [
                pltpu.VMEM((16,), f32),
                pltpu.VMEM_SHARED((n, d), f32),
                pltpu.VMEM_SHARED((n, 16), f32),
            ]
            + [pltpu.SemaphoreType.DMA] * 10
        ),
    )
    o2, s2 = edge_kernel(src, dst, h_mat, tsrc, tdst, mpat)

    y = pl.pallas_call(
        _epilogue_body,
        grid=(grid,),
        in_specs=[
            pl.BlockSpec((2, bn, d), lambda i: (0, i, 0)),
            pl.BlockSpec((2, bn, 16), lambda i: (0, i, 0)),
            pl.BlockSpec((16, d), lambda i: (0, 0)),
            pl.BlockSpec((d, out_dim), lambda i: (0, 0)),
            pl.BlockSpec((1, out_dim), lambda i: (0, 0)),
        ],
        out_specs=pl.BlockSpec((bn, out_dim), lambda i: (i, 0)),
        out_shape=jax.ShapeDtypeStruct((n, out_dim), f32),
    )(o2, s2, bexp, Wo, bo2)
    return y


# speedup vs baseline: 170.4594x; 1.4218x over previous
import functools

import jax
import jax.numpy as jnp
from jax import lax
from jax.experimental import pallas as pl
from jax.experimental.pallas import tpu as pltpu
from jax.experimental.pallas import tpu_sc as plsc

_NC = 2
_NS = 16
_C = 80


def _proj_body(x_ref, wc_ref, asrc_ref, adst_ref, h_ref, ts_ref, td_ref, m_ref):
    h = jnp.dot(x_ref[...], wc_ref[...], preferred_element_type=jnp.float32)
    h_ref[...] = h
    ts = jnp.dot(h, asrc_ref[...], preferred_element_type=jnp.float32)
    td = jnp.dot(h, adst_ref[...], preferred_element_type=jnp.float32)
    ts_ref[...] = ts
    td_ref[...] = td
    blk = jnp.concatenate(
        [jnp.max(ts, axis=0)[None, :], jnp.max(td, axis=0)[None, :]], axis=0)

    @pl.when(pl.program_id(0) == 0)
    def _():
        m_ref[...] = blk

    @pl.when(pl.program_id(0) != 0)
    def _():
        m_ref[...] = jnp.maximum(m_ref[...], blk)


def _epilogue_body(o_ref, s_ref, bexp_ref, wo_ref, bo_ref, y_ref):
    o2 = o_ref[...]
    s2 = s_ref[...]
    s = s2[0] + s2[1]
    den = jnp.dot(s, bexp_ref[...], preferred_element_type=jnp.float32)
    o = (o2[0] + o2[1]) / (den + 1e-16)
    y = jnp.dot(o, wo_ref[...], preferred_element_type=jnp.float32)
    y = y + bo_ref[...]
    y_ref[...] = jnp.where(y > 0.0, y, jnp.exp(jnp.minimum(y, 0.0)) - 1.0)


def _edge_body(n_nodes, n_edges, heads,
               ei_hbm, h_hbm, ts_hbm, td_hbm, mpat_hbm,
               o_out, s_out,
               gsrc0, gdst0, p0, rows0,
               gsrc1, gdst1, p1, rows1,
               idx0, idx1, idx2, idx3,
               mpat_v, o_sh, s_sh,
               gr0, gs0, gd0, gr1, gs1, gd1, sp0, so0, sp1, so1,
               is0, is1, is2, is3):
    buf_a = (gsrc0, gdst0, p0, rows0, gr0, gs0, gd0, sp0, so0)
    buf_b = (gsrc1, gdst1, p1, rows1, gr1, gs1, gd1, sp1, so1)
    s0, s1, s2, s3 = (idx0, is0), (idx1, is1), (idx2, is2), (idx3, is3)
    p_v, rows_v = p0, rows0
    d = heads * 16
    cid = lax.axis_index("c")
    sid = lax.axis_index("s")
    wid = cid * _NS + sid
    ew = n_edges // (_NC * _NS)
    nchunk = ew // _C
    rows_main = (n_nodes // (8 * _NS)) * 8
    tail = n_nodes - _NS * rows_main

    zv = jnp.zeros((16,), jnp.float32)

    def _zb(r, _):
        def _zc(c, _):
            rows_v[r, pl.ds(c * 16, 16)] = zv
            return 0
        lax.fori_loop(0, d // 16, _zc, 0)
        p_v[r, :] = zv
        return 0

    lax.fori_loop(0, _C, _zb, 0)

    nz_full = rows_main // _C
    z_rem = rows_main - nz_full * _C
    for r in range(nz_full):
        pltpu.sync_copy(rows_v, o_sh.at[pl.ds(sid * rows_main + r * _C, _C)])
        pltpu.sync_copy(p_v, s_sh.at[pl.ds(sid * rows_main + r * _C, _C)])
    if z_rem:
        pltpu.sync_copy(rows_v.at[pl.ds(0, z_rem)],
                        o_sh.at[pl.ds(sid * rows_main + nz_full * _C, z_rem)])
        pltpu.sync_copy(p_v.at[pl.ds(0, z_rem)],
                        s_sh.at[pl.ds(sid * rows_main + nz_full * _C, z_rem)])

    @pl.when(sid == _NS - 1)
    def _():
        base = _NS * rows_main
        pltpu.sync_copy(rows_v.at[pl.ds(0, tail)], o_sh.at[pl.ds(base, tail)])
        pltpu.sync_copy(p_v.at[pl.ds(0, tail)], s_sh.at[pl.ds(base, tail)])

    pltpu.sync_copy(mpat_hbm, mpat_v)
    plsc.subcore_barrier()

    mv = mpat_v[...]

    def _issue_idx(g, iset):
        idx, isem = iset
        pltpu.async_copy(ei_hbm.at[:, pl.ds(wid * ew + g * _C, _C)], idx, isem)

    def _wait_idx(iset):
        idx, isem = iset
        pltpu.make_async_copy(ei_hbm.at[:, pl.ds(0, _C)], idx, isem).wait()

    def _issue_gather(iset, b):
        gsrc, gdst, _, rows, gr, gs, gd, _, _ = b
        idx, _ = iset
        pltpu.async_copy(h_hbm.at[idx.at[0]], rows, gr)
        pltpu.async_copy(ts_hbm.at[idx.at[0]], gsrc, gs)
        pltpu.async_copy(td_hbm.at[idx.at[1]], gdst, gd)

    def _wait_gather(iset, b):
        gsrc, gdst, _, rows, gr, gs, gd, _, _ = b
        idx, _ = iset
        pltpu.make_async_copy(h_hbm.at[idx.at[0]], rows, gr).wait()
        pltpu.make_async_copy(ts_hbm.at[idx.at[0]], gsrc, gs).wait()
        pltpu.make_async_copy(td_hbm.at[idx.at[1]], gdst, gd).wait()

    def _issue_scatter(iset, b):
        _, _, p, rows, _, _, _, sp, so = b
        idx, _ = iset
        pltpu.async_copy(p, s_sh.at[idx.at[1]], sp, add=True)
        pltpu.async_copy(rows, o_sh.at[idx.at[1]], so, add=True)

    def _wait_scatter(iset, b):
        _, _, p, rows, _, _, _, sp, so = b
        idx, _ = iset
        pltpu.make_async_copy(p, s_sh.at[idx.at[1]], sp).wait()
        pltpu.make_async_copy(rows, o_sh.at[idx.at[1]], so).wait()

    def _compute(b):
        gsrc, gdst, p, rows, _, _, _, _, _ = b

        def _pb(i, _):
            e = gsrc[i, :] + gdst[i, :]
            e = jnp.where(e < 0.0, e * 0.2, e)
            p[i, :] = jnp.exp(e - mv)
            return 0

        lax.fori_loop(0, _C, _pb, 0)

        def _sb(i, _):
            pv = p[i, :]
            for h in range(heads):
                bc = lax.broadcast(pv[h], (16,))
                rows[i, pl.ds(h * 16, 16)] = rows[i, pl.ds(h * 16, 16)] * bc
            return 0

        lax.fori_loop(0, _C, _sb, 0)

    def _half(g, x, y, i_cur, i_nxt, i_prev):
        @pl.when(g > 0)
        def _():
            _wait_scatter(i_prev, y)

        @pl.when(g + 3 < nchunk)
        def _():
            _issue_idx(g + 3, i_prev)

        @pl.when(g + 1 < nchunk)
        def _():
            _wait_idx(i_nxt)
            _issue_gather(i_nxt, y)

        _wait_gather(i_cur, x)
        _compute(x)
        _issue_scatter(i_cur, x)

    assert nchunk % 4 == 1 and nchunk >= 5
    _issue_idx(0, s0)
    _issue_idx(1, s1)
    _issue_idx(2, s2)
    _wait_idx(s0)
    _issue_gather(s0, buf_a)

    def _body(i, _):
        g = 4 * i
        _half(g, buf_a, buf_b, s0, s1, s3)
        _half(g + 1, buf_b, buf_a, s1, s2, s0)
        _half(g + 2, buf_a, buf_b, s2, s3, s1)
        _half(g + 3, buf_b, buf_a, s3, s0, s2)
        return 0

    lax.fori_loop(0, nchunk // 4, _body, 0)
    _wait_scatter(s3, buf_b)
    _wait_gather(s0, buf_a)
    _compute(buf_a)
    _issue_scatter(s0, buf_a)
    _wait_scatter(s0, buf_a)
    plsc.subcore_barrier()

    row0 = sid * rows_main
    pltpu.sync_copy(o_sh.at[pl.ds(row0, rows_main)],
                    o_out.at[cid, pl.ds(row0, rows_main)])
    pltpu.sync_copy(s_sh.at[pl.ds(row0, rows_main)],
                    s_out.at[cid, pl.ds(row0, rows_main)])

    @pl.when(sid == _NS - 1)
    def _():
        base = _NS * rows_main
        pltpu.sync_copy(o_sh.at[pl.ds(base, tail)],
                        o_out.at[cid, pl.ds(base, tail)])
        pltpu.sync_copy(s_sh.at[pl.ds(base, tail)],
                        s_out.at[cid, pl.ds(base, tail)])


def kernel(x, edge_index, W, a, Wo, bo):
    n, din = x.shape
    heads, _, hid = W.shape
    e = edge_index.shape[1]
    d = heads * hid
    out_dim = Wo.shape[1]
    f32 = jnp.float32

    wc = jnp.transpose(W, (1, 0, 2)).reshape(din, d)
    eye = jnp.eye(heads, dtype=f32)
    a_src = (a[:, :hid][:, :, None] * eye[:, None, :]).reshape(d, heads)
    a_dst = (a[:, hid:][:, :, None] * eye[:, None, :]).reshape(d, heads)
    asrc2 = jnp.concatenate([a_src, a_src], axis=1)
    adst2 = jnp.concatenate([a_dst, a_dst], axis=1)
    bexp = jnp.concatenate(
        [jnp.kron(eye, jnp.ones((1, hid), f32)), jnp.zeros((heads, d), f32)],
        axis=0)
    bo2 = bo.reshape(1, out_dim)

    bn = 1000
    grid = n // bn
    h_mat, tsrc, tdst, mstats = pl.pallas_call(
        _proj_body,
        grid=(grid,),
        in_specs=[
            pl.BlockSpec((bn, din), lambda i: (i, 0)),
            pl.BlockSpec((din, d), lambda i: (0, 0)),
            pl.BlockSpec((d, 16), lambda i: (0, 0)),
            pl.BlockSpec((d, 16), lambda i: (0, 0)),
        ],
        out_specs=[
            pl.BlockSpec((bn, d), lambda i: (i, 0)),
            pl.BlockSpec((bn, 16), lambda i: (i, 0)),
            pl.BlockSpec((bn, 16), lambda i: (i, 0)),
            pl.BlockSpec((2, 16), lambda i: (0, 0)),
        ],
        out_shape=[
            jax.ShapeDtypeStruct((n, d), f32),
            jax.ShapeDtypeStruct((n, 16), f32),
            jax.ShapeDtypeStruct((n, 16), f32),
            jax.ShapeDtypeStruct((2, 16), f32),
        ],
    )(x, wc, asrc2, adst2)

    msum = mstats[0] + mstats[1]
    mpat = jnp.where(msum > 0.0, msum, 0.2 * msum)

    edge_kernel = pl.kernel(
        functools.partial(_edge_body, n, e, heads),
        out_type=[
            jax.ShapeDtypeStruct((_NC, n, d), f32),
            jax.ShapeDtypeStruct((_NC, n, 16), f32),
        ],
        mesh=plsc.VectorSubcoreMesh(core_axis_name="c", subcore_axis_name="s"),
        compiler_params=pltpu.CompilerParams(use_tc_tiling_on_sc=False),
        scratch_types=(
            [
                pltpu.VMEM((_C, 16), f32),
                pltpu.VMEM((_C, 16), f32),
                pltpu.VMEM((_C, 16), f32),
                pltpu.VMEM((_C, d), f32),
            ] * 2
            + [pltpu.VMEM((2, _C), jnp.int32)] * 4
            + [
                pltpu.VMEM((16,), f32),
                pltpu.VMEM_SHARED((n, d), f32),
                pltpu.VMEM_SHARED((n, 16), f32),
            ]
            + [pltpu.SemaphoreType.DMA] * 14
        ),
    )
    o2, s2 = edge_kernel(edge_index, h_mat, tsrc, tdst, mpat)

    y = pl.pallas_call(
        _epilogue_body,
        grid=(grid,),
        in_specs=[
            pl.BlockSpec((2, bn, d), lambda i: (0, i, 0)),
            pl.BlockSpec((2, bn, 16), lambda i: (0, i, 0)),
            pl.BlockSpec((16, d), lambda i: (0, 0)),
            pl.BlockSpec((d, out_dim), lambda i: (0, 0)),
            pl.BlockSpec((1, out_dim), lambda i: (0, 0)),
        ],
        out_specs=pl.BlockSpec((bn, out_dim), lambda i: (i, 0)),
        out_shape=jax.ShapeDtypeStruct((n, out_dim), f32),
    )(o2, s2, bexp, Wo, bo2)
    return y
